# baseline (speedup 1.0000x reference)
import jax
import jax.numpy as jnp
from jax.experimental import pallas as pl
from jax.experimental.pallas import tpu as pltpu

D_MODEL = 2048
D_FF = 8192
N_BLK = 32
D1 = D_FF // N_BLK
N_TOK = 4096

T_TOK = 1024
T_X = 16
C = T_X * N_BLK


def _ff_kernel(qm_ref, x_ref, w1_ref, w2_ref, b2_ref, oh_ref, res_ref):
    j = pl.program_id(1)

    mid = jnp.dot(x_ref[...], w1_ref[...], preferred_element_type=jnp.float32)

    xi = jax.lax.broadcasted_iota(jnp.int32, (D1, C), 0)
    ci = jax.lax.broadcasted_iota(jnp.int32, (D1, C), 1)
    expand = (xi == j * T_X + ci // N_BLK).astype(jnp.bfloat16)
    qm_f = qm_ref[...].astype(jnp.bfloat16)
    qm_exp = jnp.dot(qm_f, expand, preferred_element_type=jnp.float32)

    yi = jax.lax.broadcasted_iota(jnp.int32, (T_TOK, C), 1) % N_BLK
    sel = qm_exp == yi.astype(jnp.float32)

    oh_ref[...] = sel.astype(jnp.float32)

    relu = jnp.where(sel, jnp.maximum(mid, 0.0), 0.0).astype(jnp.bfloat16)

    part = jnp.dot(relu, w2_ref[...], preferred_element_type=jnp.float32)

    @pl.when(j == 0)
    def _():
        res_ref[...] = b2_ref[...] + part

    @pl.when(j != 0)
    def _():
        res_ref[...] = res_ref[...] + part


def kernel(quant_mask, mask, x, w1, w2, b2):
    del mask
    x_b = x.astype(jnp.bfloat16)
    w1_b = w1.reshape(D_MODEL, D_FF).astype(jnp.bfloat16)
    w2_b = w2.transpose(1, 0, 2).reshape(D_FF, D_MODEL).astype(jnp.bfloat16)
    b2_2d = b2.reshape(1, D_MODEL)

    grid = (N_TOK // T_TOK, D_FF // C)
    oh_flat, res = pl.pallas_call(
        _ff_kernel,
        grid=grid,
        in_specs=[
            pl.BlockSpec((T_TOK, D1), lambda i, j: (i, 0)),
            pl.BlockSpec((T_TOK, D_MODEL), lambda i, j: (i, 0)),
            pl.BlockSpec((D_MODEL, C), lambda i, j: (0, j)),
            pl.BlockSpec((C, D_MODEL), lambda i, j: (j, 0)),
            pl.BlockSpec((1, D_MODEL), lambda i, j: (0, 0)),
        ],
        out_specs=[
            pl.BlockSpec((T_TOK, C), lambda i, j: (i, j)),
            pl.BlockSpec((T_TOK, D_MODEL), lambda i, j: (i, 0)),
        ],
        out_shape=[
            jax.ShapeDtypeStruct((N_TOK, D_FF), jnp.float32),
            jax.ShapeDtypeStruct((N_TOK, D_MODEL), jnp.float32),
        ],
        compiler_params=pltpu.CompilerParams(
            dimension_semantics=("parallel", "arbitrary"),
        ),
    )(quant_mask, x_b, w1_b, w2_b, b2_2d)

    return (oh_flat.reshape(N_TOK, D1, N_BLK), res)


# speedup vs baseline: 1.0141x; 1.0141x over previous
import jax
import jax.numpy as jnp
from jax.experimental import pallas as pl
from jax.experimental.pallas import tpu as pltpu

D_MODEL = 2048
D_FF = 8192
N_BLK = 32
D1 = D_FF // N_BLK
N_TOK = 4096

T_TOK = 1024
T_X = 16
C = T_X * N_BLK


def _w2t_kernel(w2_ref, out_ref):
    blk = w2_ref[...]
    out_ref[...] = (
        jnp.transpose(blk, (1, 0, 2)).reshape(C, D_MODEL).astype(jnp.bfloat16)
    )


def _ff_kernel(qm_ref, x_ref, w1_ref, w2_ref, b2_ref, oh_ref, res_ref):
    j = pl.program_id(1)

    mid = jnp.dot(x_ref[...], w1_ref[...], preferred_element_type=jnp.float32)

    xi = jax.lax.broadcasted_iota(jnp.int32, (D1, C), 0)
    ci = jax.lax.broadcasted_iota(jnp.int32, (D1, C), 1)
    expand = (xi == j * T_X + ci // N_BLK).astype(jnp.bfloat16)
    qm_f = qm_ref[...].astype(jnp.bfloat16)
    qm_exp = jnp.dot(qm_f, expand, preferred_element_type=jnp.float32)

    yi = jax.lax.broadcasted_iota(jnp.int32, (T_TOK, C), 1) % N_BLK
    sel = qm_exp == yi.astype(jnp.float32)

    oh_ref[...] = sel.astype(jnp.float32)

    zero = jnp.zeros((), jnp.bfloat16)
    relu = jnp.where(sel, jnp.maximum(mid, 0.0).astype(jnp.bfloat16), zero)

    part = jnp.dot(relu, w2_ref[...], preferred_element_type=jnp.float32)

    @pl.when(j == 0)
    def _():
        res_ref[...] = b2_ref[...] + part

    @pl.when(j != 0)
    def _():
        res_ref[...] = res_ref[...] + part


def kernel(quant_mask, mask, x, w1, w2, b2):
    del mask
    x_b = x.astype(jnp.bfloat16)
    w1_b = w1.reshape(D_MODEL, D_FF).astype(jnp.bfloat16)
    b2_2d = b2.reshape(1, D_MODEL)

    w2_b = pl.pallas_call(
        _w2t_kernel,
        grid=(D_FF // C,),
        in_specs=[pl.BlockSpec((N_BLK, T_X, D_MODEL), lambda j: (0, j, 0))],
        out_specs=pl.BlockSpec((C, D_MODEL), lambda j: (j, 0)),
        out_shape=jax.ShapeDtypeStruct((D_FF, D_MODEL), jnp.bfloat16),
    )(w2)

    grid = (N_TOK // T_TOK, D_FF // C)
    oh_flat, res = pl.pallas_call(
        _ff_kernel,
        grid=grid,
        in_specs=[
            pl.BlockSpec((T_TOK, D1), lambda i, j: (i, 0)),
            pl.BlockSpec((T_TOK, D_MODEL), lambda i, j: (i, 0)),
            pl.BlockSpec((D_MODEL, C), lambda i, j: (0, j)),
            pl.BlockSpec((C, D_MODEL), lambda i, j: (j, 0)),
            pl.BlockSpec((1, D_MODEL), lambda i, j: (0, 0)),
        ],
        out_specs=[
            pl.BlockSpec((T_TOK, C), lambda i, j: (i, j)),
            pl.BlockSpec((T_TOK, D_MODEL), lambda i, j: (i, 0)),
        ],
        out_shape=[
            jax.ShapeDtypeStruct((N_TOK, D_FF), jnp.float32),
            jax.ShapeDtypeStruct((N_TOK, D_MODEL), jnp.float32),
        ],
        compiler_params=pltpu.CompilerParams(
            dimension_semantics=("parallel", "arbitrary"),
        ),
    )(quant_mask, x_b, w1_b, w2_b, b2_2d)

    return (oh_flat.reshape(N_TOK, D1, N_BLK), res)
